# Initial kernel scaffold; baseline (speedup 1.0000x reference)
import functools

import jax
import jax.numpy as jnp
from jax import lax
from jax.experimental import pallas as pl
from jax.experimental.pallas import tpu as pltpu
from jax.experimental.pallas import tpu_sc as plsc

N = 50000
D = 256
G = 128
NC = 2
NS = 16
NW = NC * NS
C = 1600
S = 80


def _phase1_body(x_hbm, ids_hbm, zeros_hbm, part_hbm,
                 buf_a, buf_b, idx_a, idx_b, acc,
                 sem_a, sem_b, sem_ia, sem_ib, sem_z):
    wid = lax.axis_index("s") * NC + lax.axis_index("c")
    base = wid * C
    nit = jnp.minimum(N - base, C) // S

    def start(i, buf, ibuf, rsem, isem):
        off = base + i * S
        pltpu.make_async_copy(x_hbm.at[pl.ds(off, S)], buf, rsem).start()
        pltpu.make_async_copy(ids_hbm.at[pl.ds(off, S)], ibuf, isem).start()

    start(0, buf_a, idx_a, sem_a, sem_ia)
    start(1, buf_b, idx_b, sem_b, sem_ib)
    pltpu.make_async_copy(zeros_hbm, acc, sem_z).start()
    pltpu.make_async_copy(zeros_hbm, acc, sem_z).wait()

    bufs = ((buf_a, idx_a, sem_a, sem_ia), (buf_b, idx_b, sem_b, sem_ib))

    def body(k, carry):
        for b in range(2):
            buf, ibuf, rsem, isem = bufs[b]
            i = 2 * k + b

            @pl.when(i < nit)
            def _():
                off = base + i * S
                pltpu.make_async_copy(x_hbm.at[pl.ds(off, S)], buf, rsem).wait()
                pltpu.make_async_copy(ids_hbm.at[pl.ds(off, S)], ibuf, isem).wait()
                pltpu.sync_copy(buf, acc.at[ibuf], add=True)

                @pl.when(i + 2 < nit)
                def _():
                    start(i + 2, buf, ibuf, rsem, isem)
        return carry

    lax.fori_loop(0, (nit + 1) // 2, body, 0)
    pltpu.sync_copy(acc, part_hbm.at[wid])


_phase1 = functools.partial(
    pl.kernel,
    out_type=jax.ShapeDtypeStruct((NW, G, D), jnp.float32),
    mesh=plsc.VectorSubcoreMesh(core_axis_name="c", subcore_axis_name="s"),
    scratch_types=[
        pltpu.VMEM((S, D), jnp.float32),
        pltpu.VMEM((S, D), jnp.float32),
        pltpu.VMEM((S,), jnp.int32),
        pltpu.VMEM((S,), jnp.int32),
        pltpu.VMEM((G, D), jnp.float32),
        pltpu.SemaphoreType.DMA,
        pltpu.SemaphoreType.DMA,
        pltpu.SemaphoreType.DMA,
        pltpu.SemaphoreType.DMA,
        pltpu.SemaphoreType.DMA,
    ],
)(_phase1_body)


def _reduce_body(p_ref, o_ref):
    o_ref[...] = jnp.sum(p_ref[...], axis=0)


def _phase2(partials):
    return pl.pallas_call(
        _reduce_body,
        out_shape=jax.ShapeDtypeStruct((G, D), jnp.float32),
    )(partials)


@jax.jit
def kernel(node_feature, segment_ids, num_graphs):
    ids = segment_ids.astype(jnp.int32)
    zeros = jnp.zeros((G, D), jnp.float32)
    partials = _phase1(node_feature, ids, zeros)
    return _phase2(partials)


# speedup vs baseline: 2.4613x; 2.4613x over previous
import functools

import jax
import jax.numpy as jnp
from jax import lax
from jax.experimental import pallas as pl
from jax.experimental.pallas import tpu as pltpu
from jax.experimental.pallas import tpu_sc as plsc

N = 50000
D = 256
G = 128
NC = 2
NS = 16
NW = NC * NS
C = 1600
S = 80
L = 16


def _phase1_body(x_hbm, ids_hbm, zeros_hbm, part_hbm,
                 buf_a, buf_b, idx_a, idx_b, acc,
                 sem_a, sem_b, sem_ia, sem_ib, sem_z):
    cid = lax.axis_index("c")
    sid = lax.axis_index("s")
    wid = sid * NC + cid
    base = wid * C
    nit = jnp.minimum(N - base, C) // S

    def start(i, buf, ibuf, rsem, isem):
        off = base + i * S
        pltpu.make_async_copy(x_hbm.at[pl.ds(off * D, S * D)], buf, rsem).start()
        pltpu.make_async_copy(ids_hbm.at[pl.ds(off, S)], ibuf, isem).start()

    start(0, buf_a, idx_a, sem_a, sem_ia)
    start(1, buf_b, idx_b, sem_b, sem_ib)
    pltpu.make_async_copy(zeros_hbm, acc, sem_z).start()
    pltpu.make_async_copy(zeros_hbm, acc, sem_z).wait()

    bufs = ((buf_a, idx_a, sem_a, sem_ia), (buf_b, idx_b, sem_b, sem_ib))

    def body(k, carry):
        for b in range(2):
            buf, ibuf, rsem, isem = bufs[b]
            i = 2 * k + b

            @pl.when(i < nit)
            def _():
                off = base + i * S
                pltpu.make_async_copy(
                    x_hbm.at[pl.ds(off * D, S * D)], buf, rsem).wait()
                pltpu.make_async_copy(
                    ids_hbm.at[pl.ds(off, S)], ibuf, isem).wait()

                def group_body(g, gcarry):
                    idv = ibuf[pl.ds(g * L, L)]
                    for r in range(L):
                        rbase = idv[r] * D
                        row = (g * L + r) * D
                        for j in range(D // L):
                            v = buf[pl.ds(row + L * j, L)]
                            plsc.addupdate(acc.at[pl.ds(rbase + L * j, L)], v)
                    return gcarry

                lax.fori_loop(0, S // L, group_body, 0)

                @pl.when(i + 2 < nit)
                def _():
                    start(i + 2, buf, ibuf, rsem, isem)
        return carry

    lax.fori_loop(0, (nit + 1) // 2, body, 0)
    pltpu.sync_copy(acc, part_hbm.at[pl.ds(wid * G * D, G * D)])


_phase1 = functools.partial(
    pl.kernel,
    out_type=jax.ShapeDtypeStruct((NW * G * D,), jnp.float32),
    mesh=plsc.VectorSubcoreMesh(core_axis_name="c", subcore_axis_name="s"),
    scratch_types=[
        pltpu.VMEM((S * D,), jnp.float32),
        pltpu.VMEM((S * D,), jnp.float32),
        pltpu.VMEM((S,), jnp.int32),
        pltpu.VMEM((S,), jnp.int32),
        pltpu.VMEM((G * D,), jnp.float32),
        pltpu.SemaphoreType.DMA,
        pltpu.SemaphoreType.DMA,
        pltpu.SemaphoreType.DMA,
        pltpu.SemaphoreType.DMA,
        pltpu.SemaphoreType.DMA,
    ],
)(_phase1_body)


def _reduce_body(p_ref, o_ref):
    o_ref[...] = jnp.sum(p_ref[...], axis=0)


def _phase2(partials):
    return pl.pallas_call(
        _reduce_body,
        out_shape=jax.ShapeDtypeStruct((G, D), jnp.float32),
    )(partials)


@jax.jit
def kernel(node_feature, segment_ids, num_graphs):
    ids = segment_ids.astype(jnp.int32)
    zeros = jnp.zeros((G * D,), jnp.float32)
    partials = _phase1(node_feature.reshape(N * D), ids, zeros)
    return _phase2(partials.reshape(NW, G, D))


# speedup vs baseline: 4.7868x; 1.9448x over previous
import functools

import jax
import jax.numpy as jnp
from jax import lax
from jax.experimental import pallas as pl
from jax.experimental.pallas import tpu as pltpu
from jax.experimental.pallas import tpu_sc as plsc

N = 50000
D = 256
G = 128
NC = 2
NS = 16
NW = NC * NS
C = 1600
S = 80
L = 16


def _phase1_body(x_hbm, ids_hbm, zeros_hbm, part_hbm,
                 buf_a, buf_b, idx_a, idx_b, acc,
                 sem_a, sem_b, sem_ia, sem_ib, sem_z):
    cid = lax.axis_index("c")
    sid = lax.axis_index("s")
    wid = sid * NC + cid
    base = wid * C
    nit = jnp.minimum(N - base, C) // S

    def start(i, buf, ibuf, rsem, isem):
        off = base + i * S
        pltpu.make_async_copy(x_hbm.at[pl.ds(off, S)], buf, rsem).start()
        pltpu.make_async_copy(ids_hbm.at[pl.ds(off, S)], ibuf, isem).start()

    start(0, buf_a, idx_a, sem_a, sem_ia)
    start(1, buf_b, idx_b, sem_b, sem_ib)
    pltpu.make_async_copy(zeros_hbm, acc, sem_z).start()
    pltpu.make_async_copy(zeros_hbm, acc, sem_z).wait()

    bufs = ((buf_a, idx_a, sem_a, sem_ia), (buf_b, idx_b, sem_b, sem_ib))

    def body(k, carry):
        for b in range(2):
            buf, ibuf, rsem, isem = bufs[b]
            i = 2 * k + b

            @pl.when(i < nit)
            def _():
                off = base + i * S
                pltpu.make_async_copy(
                    x_hbm.at[pl.ds(off, S)], buf, rsem).wait()
                pltpu.make_async_copy(
                    ids_hbm.at[pl.ds(off, S)], ibuf, isem).wait()

                def group_body(g, gcarry):
                    idv = ibuf[pl.ds(g * L, L)]
                    seg0 = idv[0]
                    uniform = seg0 == idv[L - 1]

                    @pl.when(uniform)
                    def _():
                        gbase = seg0 * D
                        for j in range(D // L):
                            v = buf[g * L, pl.ds(L * j, L)]
                            for r in range(1, L):
                                v = v + buf[g * L + r, pl.ds(L * j, L)]
                            plsc.addupdate(acc.at[pl.ds(gbase + L * j, L)], v)

                    @pl.when(jnp.logical_not(uniform))
                    def _():
                        for r in range(L):
                            rbase = idv[r] * D
                            for j in range(D // L):
                                v = buf[g * L + r, pl.ds(L * j, L)]
                                plsc.addupdate(
                                    acc.at[pl.ds(rbase + L * j, L)], v)

                    return gcarry

                lax.fori_loop(0, S // L, group_body, 0)

                @pl.when(i + 2 < nit)
                def _():
                    start(i + 2, buf, ibuf, rsem, isem)
        return carry

    lax.fori_loop(0, (nit + 1) // 2, body, 0)
    pltpu.sync_copy(acc, part_hbm.at[pl.ds(wid * G * D, G * D)])


_phase1 = functools.partial(
    pl.kernel,
    out_type=jax.ShapeDtypeStruct((NW * G * D,), jnp.float32),
    mesh=plsc.VectorSubcoreMesh(core_axis_name="c", subcore_axis_name="s"),
    scratch_types=[
        pltpu.VMEM((S, D), jnp.float32),
        pltpu.VMEM((S, D), jnp.float32),
        pltpu.VMEM((S,), jnp.int32),
        pltpu.VMEM((S,), jnp.int32),
        pltpu.VMEM((G * D,), jnp.float32),
        pltpu.SemaphoreType.DMA,
        pltpu.SemaphoreType.DMA,
        pltpu.SemaphoreType.DMA,
        pltpu.SemaphoreType.DMA,
        pltpu.SemaphoreType.DMA,
    ],
)(_phase1_body)


def _reduce_body(p_ref, o_ref):
    o_ref[...] = jnp.sum(p_ref[...], axis=0)


def _phase2(partials):
    return pl.pallas_call(
        _reduce_body,
        out_shape=jax.ShapeDtypeStruct((G, D), jnp.float32),
    )(partials)


@jax.jit
def kernel(node_feature, segment_ids, num_graphs):
    ids = segment_ids.astype(jnp.int32)
    zeros = jnp.zeros((G * D,), jnp.float32)
    partials = _phase1(node_feature, ids, zeros)
    return _phase2(partials.reshape(NW, G, D))


# speedup vs baseline: 5.6811x; 1.1868x over previous
import functools

import jax
import jax.numpy as jnp
from jax import lax
from jax.experimental import pallas as pl
from jax.experimental.pallas import tpu as pltpu
from jax.experimental.pallas import tpu_sc as plsc

N = 50000
D = 256
G = 128
NC = 2
NS = 16
NW = NC * NS
C = 1600
S = 80
L = 16


def _phase1_body(x_hbm, ids_hbm, zeros_hbm, part_hbm,
                 buf_a, buf_b, idx_a, idx_b, acc,
                 sem_a, sem_b, sem_ia, sem_ib, sem_z):
    cid = lax.axis_index("c")
    sid = lax.axis_index("s")
    wid = sid * NC + cid
    base = wid * C
    nit = jnp.minimum(N - base, C) // S

    def start(i, buf, ibuf, rsem, isem):
        off = base + i * S
        pltpu.make_async_copy(x_hbm.at[pl.ds(off, S)], buf, rsem).start()
        pltpu.make_async_copy(ids_hbm.at[pl.ds(off, S)], ibuf, isem).start()

    start(0, buf_a, idx_a, sem_a, sem_ia)
    start(1, buf_b, idx_b, sem_b, sem_ib)
    pltpu.make_async_copy(zeros_hbm, acc, sem_z).start()
    pltpu.make_async_copy(zeros_hbm, acc, sem_z).wait()

    bufs = ((buf_a, idx_a, sem_a, sem_ia), (buf_b, idx_b, sem_b, sem_ib))

    def body(k, carry):
        for b in range(2):
            buf, ibuf, rsem, isem = bufs[b]
            i = 2 * k + b

            @pl.when(i < nit)
            def _():
                off = base + i * S
                pltpu.make_async_copy(
                    x_hbm.at[pl.ds(off, S)], buf, rsem).wait()
                pltpu.make_async_copy(
                    ids_hbm.at[pl.ds(off, S)], ibuf, isem).wait()

                def group_body(g, gcarry):
                    idv = ibuf[pl.ds(g * L, L)]
                    seg0 = idv[0]
                    uniform = seg0 == idv[L - 1]

                    @pl.when(uniform)
                    def _():
                        gbase = seg0 * D
                        for j in range(D // L):
                            vs = [buf[g * L + r, pl.ds(L * j, L)]
                                  for r in range(L)]
                            while len(vs) > 1:
                                vs = [vs[t] + vs[t + 1]
                                      for t in range(0, len(vs), 2)]
                            plsc.addupdate(acc.at[pl.ds(gbase + L * j, L)],
                                           vs[0])

                    @pl.when(jnp.logical_not(uniform))
                    def _():
                        for r in range(L):
                            rbase = idv[r] * D
                            for j in range(D // L):
                                v = buf[g * L + r, pl.ds(L * j, L)]
                                plsc.addupdate(
                                    acc.at[pl.ds(rbase + L * j, L)], v)

                    return gcarry

                lax.fori_loop(0, S // L, group_body, 0)

                @pl.when(i + 2 < nit)
                def _():
                    start(i + 2, buf, ibuf, rsem, isem)
        return carry

    lax.fori_loop(0, (nit + 1) // 2, body, 0)
    pltpu.sync_copy(acc, part_hbm.at[pl.ds(wid * G * D, G * D)])


_phase1 = functools.partial(
    pl.kernel,
    out_type=jax.ShapeDtypeStruct((NW * G * D,), jnp.float32),
    mesh=plsc.VectorSubcoreMesh(core_axis_name="c", subcore_axis_name="s"),
    scratch_types=[
        pltpu.VMEM((S, D), jnp.float32),
        pltpu.VMEM((S, D), jnp.float32),
        pltpu.VMEM((S,), jnp.int32),
        pltpu.VMEM((S,), jnp.int32),
        pltpu.VMEM((G * D,), jnp.float32),
        pltpu.SemaphoreType.DMA,
        pltpu.SemaphoreType.DMA,
        pltpu.SemaphoreType.DMA,
        pltpu.SemaphoreType.DMA,
        pltpu.SemaphoreType.DMA,
    ],
)(_phase1_body)


def _reduce_body(p_ref, o_ref):
    o_ref[...] = jnp.sum(p_ref[...], axis=0)


def _phase2(partials):
    return pl.pallas_call(
        _reduce_body,
        out_shape=jax.ShapeDtypeStruct((G, D), jnp.float32),
    )(partials)


@jax.jit
def kernel(node_feature, segment_ids, num_graphs):
    ids = segment_ids.astype(jnp.int32)
    zeros = jnp.zeros((G * D,), jnp.float32)
    partials = _phase1(node_feature, ids, zeros)
    return _phase2(partials.reshape(NW, G, D))
